# baseline (speedup 1.0000x reference)
import functools

import jax
import jax.numpy as jnp
from jax import lax
from jax.experimental import pallas as pl
from jax.experimental.pallas import tpu as pltpu
from jax.experimental.pallas import tpu_sc as plsc

_B = 16384
_D = 32
_NC = 2
_NS = 16
_NW = _NC * _NS
_BPW = _B // _NW
_CH = 128
_NCHUNK = _BPW // _CH


def _body(idx_hbm, sp_hbm, su_hbm, out_sp, out_su, idx_v, rows_sp, rows_su, sem):
    wid = lax.axis_index("s") * _NC + lax.axis_index("c")
    base = wid * _BPW
    for c in range(_NCHUNK):
        pltpu.sync_copy(idx_hbm.at[pl.ds(base + c * _CH, _CH)], idx_v.at[c])
    copies = []
    for c in range(_NCHUNK):
        copies.append(
            pltpu.async_copy(sp_hbm.at[idx_v.at[c]], rows_sp.at[pl.ds(c * _CH, _CH)], sem))
        copies.append(
            pltpu.async_copy(su_hbm.at[idx_v.at[c]], rows_su.at[pl.ds(c * _CH, _CH)], sem))
    for cp in copies:
        cp.wait()
    pltpu.sync_copy(rows_sp, out_sp.at[pl.ds(base, _BPW)])
    pltpu.sync_copy(rows_su, out_su.at[pl.ds(base, _BPW)])


@jax.jit
def kernel(node_indices, B_sp, B_su):
    gather = pl.kernel(
        _body,
        out_type=(
            jax.ShapeDtypeStruct((_B, _D), jnp.float32),
            jax.ShapeDtypeStruct((_B, _D), jnp.float32),
        ),
        mesh=plsc.VectorSubcoreMesh(core_axis_name="c", subcore_axis_name="s"),
        scratch_types=[
            pltpu.VMEM((_NCHUNK, _CH), jnp.int32),
            pltpu.VMEM((_BPW, _D), jnp.float32),
            pltpu.VMEM((_BPW, _D), jnp.float32),
            pltpu.SemaphoreType.DMA,
        ],
        compiler_params=pltpu.CompilerParams(use_tc_tiling_on_sc=False),
    )
    return gather(node_indices.astype(jnp.int32), B_sp, B_su)
